# baseline (speedup 1.0000x reference)
import functools

import jax
import jax.numpy as jnp
from jax import lax
from jax.experimental import pallas as pl
from jax.experimental.pallas import tpu as pltpu
from jax.experimental.pallas import tpu_sc as plsc

L = 16
NC, NS = 2, 16
NW = NC * NS
UNROLL = 8


def _pooled_spans(h2, attn1, spans1, gidx1, B, S, M, W, D):
    BM = B * M
    spans_per_w = BM // NW
    chunks = D // L
    mesh = plsc.VectorSubcoreMesh(core_axis_name="c", subcore_axis_name="s")

    @functools.partial(
        pl.kernel,
        out_type=jax.ShapeDtypeStruct((BM, D), jnp.float32),
        mesh=mesh,
        scratch_types=[
            pltpu.VMEM((BM // NW * W,), jnp.int32),
            pltpu.VMEM((BM // NW * W,), jnp.int32),
            pltpu.VMEM((2, W, D), jnp.float32),
            pltpu.VMEM((D,), jnp.float32),
            pltpu.VMEM((W * L,), jnp.float32),
            pltpu.VMEM((2 * W,), jnp.float32),
            pltpu.VMEM((spans_per_w, D), jnp.float32),
            pltpu.SemaphoreType.DMA,
            pltpu.SemaphoreType.DMA,
        ],
        compiler_params=pltpu.CompilerParams(needs_layout_passes=False),
    )
    def run(h_hbm, attn_hbm, spans_hbm, gidx_hbm, out_hbm,
            raw_v, idx_v, rows_v, aw_v, part_v, wts_v, outb_v, sem0, sem1):
        wid = lax.axis_index("s") * NC + lax.axis_index("c")
        r0 = wid * spans_per_w
        pltpu.sync_copy(attn_hbm, aw_v)
        pltpu.sync_copy(spans_hbm.at[pl.ds(r0 * W, spans_per_w * W)], raw_v)
        pltpu.sync_copy(gidx_hbm.at[pl.ds(r0 * W, spans_per_w * W)], idx_v)
        lane = lax.broadcasted_iota(jnp.int32, (L,), 0)
        sems = (sem0, sem1)

        def start_gather(sp, buf):
            pltpu.async_copy(h_hbm.at[idx_v.at[pl.ds(sp * W, W)]],
                             rows_v.at[buf], sems[buf])

        def compute(sp, buf):
            rows = rows_v.at[buf]
            pltpu.make_async_copy(h_hbm.at[idx_v.at[pl.ds(sp * W, W)]], rows,
                                  sems[buf]).wait()
            maskv = raw_v[pl.ds(sp * W, W)] == -1

            accs0 = tuple(jnp.zeros((L,), jnp.float32) for _ in range(W))

            @plsc.parallel_loop(0, chunks, 1, unroll=UNROLL, carry=accs0)
            def accs(c, accs_in):
                off = pl.multiple_of(c * L, L)
                awc = aw_v[pl.ds(off, L)]
                return tuple(accs_in[w] + rows[w, pl.ds(off, L)] * awc
                             for w in range(W))
            for w in range(W):
                part_v[pl.ds(w * L, L)] = accs[w]

            logits = jnp.zeros((L,), jnp.float32)
            for j in range(L):
                logits = logits + plsc.load_gather(part_v, [lane * L + j])

            logits = jnp.where(maskv, jnp.float32(-10000.0), logits)
            e = jnp.exp(logits - jnp.max(logits))
            wts = jnp.where(maskv, jnp.float32(0.0), e / jnp.sum(e))
            wts_v[pl.ds(0, L)] = wts
            wts_v[pl.ds(L, L)] = wts

            wsplat = tuple(
                plsc.load_gather(wts_v, [jnp.full((L,), L + w, jnp.int32)])
                for w in range(W))

            @plsc.parallel_loop(0, chunks, 1, unroll=UNROLL)
            def _pool(c):
                off = pl.multiple_of(c * L, L)
                acc = rows[0, pl.ds(off, L)] * wsplat[0]
                for w in range(1, W):
                    acc = acc + rows[w, pl.ds(off, L)] * wsplat[w]
                outb_v[sp, pl.ds(off, L)] = acc

        start_gather(0, 0)

        def group_body(g, carry):
            sp0 = g * 2
            start_gather(sp0 + 1, 1)
            compute(sp0, 0)

            @pl.when(g < spans_per_w // 2 - 1)
            def _():
                start_gather(sp0 + 2, 0)

            compute(sp0 + 1, 1)
            return carry

        lax.fori_loop(0, spans_per_w // 2, group_body, 0)
        pltpu.sync_copy(outb_v, out_hbm.at[pl.ds(r0, spans_per_w)])

    return run(h2, attn1, spans1, gidx1)


def kernel(h, attn_w, spans):
    B, S, D = h.shape
    _, M, W = spans.shape
    h2 = h.reshape(B * S, D)
    attn1 = attn_w.reshape(D)
    spans1 = spans.reshape(B * M * W)
    base = (jnp.arange(B, dtype=jnp.int32) * S)[:, None, None]
    gidx1 = jnp.where(spans == -1, 0, spans + base).reshape(B * M * W)
    out = _pooled_spans(h2, attn1, spans1, gidx1, B, S, M, W, D)
    return out.reshape(B, M, D)


# speedup vs baseline: 1.1656x; 1.1656x over previous
import functools

import jax
import jax.numpy as jnp
from jax import lax
from jax.experimental import pallas as pl
from jax.experimental.pallas import tpu as pltpu
from jax.experimental.pallas import tpu_sc as plsc

L = 16
NC, NS = 2, 16
NW = NC * NS
UNROLL = 2


def _pooled_spans(h2, attn1, spans1, gidx1, B, S, M, W, D):
    BM = B * M
    spans_per_w = BM // NW
    chunks = D // L
    mesh = plsc.VectorSubcoreMesh(core_axis_name="c", subcore_axis_name="s")

    @functools.partial(
        pl.kernel,
        out_type=jax.ShapeDtypeStruct((BM, D), jnp.float32),
        mesh=mesh,
        scratch_types=[
            pltpu.VMEM((BM // NW * W,), jnp.int32),
            pltpu.VMEM((BM // NW * W,), jnp.int32),
            pltpu.VMEM((2, W, D), jnp.float32),
            pltpu.VMEM((D,), jnp.float32),
            pltpu.VMEM((W * L,), jnp.float32),
            pltpu.VMEM((2 * W,), jnp.float32),
            pltpu.VMEM((spans_per_w, D), jnp.float32),
            pltpu.SemaphoreType.DMA,
            pltpu.SemaphoreType.DMA,
        ],
        compiler_params=pltpu.CompilerParams(needs_layout_passes=False),
    )
    def run(h_hbm, attn_hbm, spans_hbm, gidx_hbm, out_hbm,
            raw_v, idx_v, rows_v, aw_v, part_v, wts_v, outb_v, sem0, sem1):
        wid = lax.axis_index("s") * NC + lax.axis_index("c")
        r0 = wid * spans_per_w
        pltpu.sync_copy(attn_hbm, aw_v)
        pltpu.sync_copy(spans_hbm.at[pl.ds(r0 * W, spans_per_w * W)], raw_v)
        pltpu.sync_copy(gidx_hbm.at[pl.ds(r0 * W, spans_per_w * W)], idx_v)
        lane = lax.broadcasted_iota(jnp.int32, (L,), 0)
        sems = (sem0, sem1)

        def start_gather(sp, buf):
            pltpu.async_copy(h_hbm.at[idx_v.at[pl.ds(sp * W, W)]],
                             rows_v.at[buf], sems[buf])

        def compute(sp, buf):
            rows = rows_v.at[buf]
            pltpu.make_async_copy(h_hbm.at[idx_v.at[pl.ds(sp * W, W)]], rows,
                                  sems[buf]).wait()
            maskv = raw_v[pl.ds(sp * W, W)] == -1

            accs0 = tuple(jnp.zeros((L,), jnp.float32) for _ in range(W))

            @plsc.parallel_loop(0, chunks, 1, unroll=UNROLL, carry=accs0)
            def accs(c, accs_in):
                off = pl.multiple_of(c * L, L)
                awc = aw_v[pl.ds(off, L)]
                return tuple(accs_in[w] + rows[w, pl.ds(off, L)] * awc
                             for w in range(W))
            for w in range(W):
                part_v[pl.ds(w * L, L)] = accs[w]

            logits = jnp.zeros((L,), jnp.float32)
            for j in range(L):
                logits = logits + plsc.load_gather(part_v, [lane * L + j])

            logits = jnp.where(maskv, jnp.float32(-10000.0), logits)
            e = jnp.exp(logits - jnp.max(logits))
            wts = jnp.where(maskv, jnp.float32(0.0), e / jnp.sum(e))
            wts_v[pl.ds(0, L)] = wts
            wts_v[pl.ds(L, L)] = wts

            wsplat = tuple(
                plsc.load_gather(wts_v, [jnp.full((L,), L + w, jnp.int32)])
                for w in range(W))

            @plsc.parallel_loop(0, chunks, 1, unroll=UNROLL)
            def _pool(c):
                off = pl.multiple_of(c * L, L)
                acc = rows[0, pl.ds(off, L)] * wsplat[0]
                for w in range(1, W):
                    acc = acc + rows[w, pl.ds(off, L)] * wsplat[w]
                outb_v[sp, pl.ds(off, L)] = acc

        start_gather(0, 0)

        def group_body(g, carry):
            sp0 = g * 2
            start_gather(sp0 + 1, 1)
            compute(sp0, 0)

            @pl.when(g < spans_per_w // 2 - 1)
            def _():
                start_gather(sp0 + 2, 0)

            compute(sp0 + 1, 1)
            return carry

        lax.fori_loop(0, spans_per_w // 2, group_body, 0)
        pltpu.sync_copy(outb_v, out_hbm.at[pl.ds(r0, spans_per_w)])

    return run(h2, attn1, spans1, gidx1)


def kernel(h, attn_w, spans):
    B, S, D = h.shape
    _, M, W = spans.shape
    h2 = h.reshape(B * S, D)
    attn1 = attn_w.reshape(D)
    spans1 = spans.reshape(B * M * W)
    base = (jnp.arange(B, dtype=jnp.int32) * S)[:, None, None]
    gidx1 = jnp.where(spans == -1, 0, spans + base).reshape(B * M * W)
    out = _pooled_spans(h2, attn1, spans1, gidx1, B, S, M, W, D)
    return out.reshape(B, M, D)
